# baseline (speedup 1.0000x reference)
import functools

import jax
import jax.numpy as jnp
from jax import lax
from jax.experimental import pallas as pl
from jax.experimental.pallas import tpu as pltpu
from jax.experimental.pallas import tpu_sc as plsc

_L = 16


@functools.lru_cache(maxsize=None)
def _spline_sc(n_q: int, n_k: int):
    info = plsc.get_sparse_core_info()
    nw = info.num_cores * info.num_subcores
    per_w = n_q // nw
    assert per_w * nw == n_q
    chunk = min(16384, per_w)
    nch = per_w // chunk
    assert nch * chunk == per_w
    unroll = 8
    nvec = chunk // _L
    assert nvec % unroll == 0
    scale = float(n_k - 1)
    npad = n_k + _L

    mesh = plsc.VectorSubcoreMesh(core_axis_name="c", subcore_axis_name="s")

    @functools.partial(
        pl.kernel,
        mesh=mesh,
        out_type=jax.ShapeDtypeStruct((n_q,), jnp.float32),
        compiler_params=pltpu.CompilerParams(needs_layout_passes=False),
        scratch_types=[
            pltpu.VMEM((npad,), jnp.float32),
            pltpu.VMEM((npad,), jnp.float32),
            pltpu.VMEM((npad,), jnp.float32),
            pltpu.VMEM((n_k,), jnp.float32),
            pltpu.VMEM((n_k,), jnp.float32),
            pltpu.VMEM((n_k,), jnp.int32),
            pltpu.VMEM((chunk,), jnp.float32),
            pltpu.VMEM((chunk,), jnp.float32),
            pltpu.VMEM((chunk,), jnp.float32),
            pltpu.VMEM((chunk,), jnp.float32),
            pltpu.SemaphoreType.DMA,
            pltpu.SemaphoreType.DMA,
            pltpu.SemaphoreType.DMA,
            pltpu.SemaphoreType.DMA,
        ],
    )
    def k(x_hbm, xp_hbm, yp_hbm, d2_hbm, out_hbm,
          xpb, ypb, d2b, t0, t1, t23, xb0, xb1, ob0, ob1,
          si0, si1, so0, so1):
        wid = lax.axis_index("s") * info.num_cores + lax.axis_index("c")
        base = wid * per_w
        xbufs, obufs = (xb0, xb1), (ob0, ob1)
        isems, osems = (si0, si1), (so0, so1)

        def in_copy(c):
            return pltpu.make_async_copy(
                x_hbm.at[pl.ds(base + c * chunk, chunk)], xbufs[c % 2],
                isems[c % 2])

        def out_copy(c):
            return pltpu.make_async_copy(
                obufs[c % 2], out_hbm.at[pl.ds(base + c * chunk, chunk)],
                osems[c % 2])

        in_copy(0).start()
        pltpu.sync_copy(xp_hbm, xpb.at[pl.ds(0, n_k)])
        pltpu.sync_copy(yp_hbm, ypb.at[pl.ds(0, n_k)])
        pltpu.sync_copy(d2_hbm, d2b.at[pl.ds(0, n_k)])

        @plsc.parallel_loop(0, n_k // _L, 1, unroll=4)
        def build(j):
            off = j * _L
            xv = xpb[pl.ds(off, _L)]
            xv1 = xpb[pl.ds(off + 1, _L)]
            yv = ypb[pl.ds(off, _L)]
            yv1 = ypb[pl.ds(off + 1, _L)]
            dv = d2b[pl.ds(off, _L)]
            dv1 = d2b[pl.ds(off + 1, _L)]
            h = xv1 - xv
            h26 = (h * h) * jnp.float32(1.0 / 6.0)
            g = h26 * dv
            t0[pl.ds(off, _L)] = yv
            t1[pl.ds(off, _L)] = (yv1 - yv) - h26 * (2.0 * dv + dv1)
            c2 = 3.0 * g
            c3 = h26 * dv1 - g
            packed = plsc.pack(c3, c2, format=plsc.PackFormat.INTERLEAVED)
            t23[pl.ds(off, _L)] = plsc.bitcast(packed, jnp.int32)

        def compute(xb, ob):
            @plsc.parallel_loop(0, nvec, 1, unroll=unroll)
            def body(j):
                off = j * _L
                t = xb[pl.ds(off, _L)] * scale
                iv = t.astype(jnp.int32)
                u = t - iv.astype(jnp.float32)
                p0 = plsc.load_gather(t0, [iv])
                p1 = plsc.load_gather(t1, [iv])
                w = plsc.load_gather(t23, [iv])
                p2 = plsc.bitcast(w & jnp.int32(-65536), jnp.float32)
                p3 = plsc.bitcast(w << 16, jnp.float32)
                ob[pl.ds(off, _L)] = p0 + u * (p1 + u * (p2 + u * p3))

        for c in range(nch):
            if c + 1 < nch:
                in_copy(c + 1).start()
            in_copy(c).wait()
            if c >= 2:
                out_copy(c - 2).wait()
            compute(xbufs[c % 2], obufs[c % 2])
            out_copy(c).start()
        if nch >= 2:
            out_copy(nch - 2).wait()
        out_copy(nch - 1).wait()

    return k


def kernel(x, x_points, y_points, d2y_points):
    return _spline_sc(x.shape[0], x_points.shape[0])(
        x, x_points, y_points, d2y_points)
